# Initial kernel scaffold; baseline (speedup 1.0000x reference)
import jax
import jax.numpy as jnp
from jax.experimental import pallas as pl


def kernel(x, prototypes, B, A, bias, temp):
    raise NotImplementedError("write your pallas kernel here")


# speedup vs baseline: 14.9787x; 14.9787x over previous
import jax
import jax.numpy as jnp
from jax.experimental import pallas as pl
from jax.experimental.pallas import tpu as pltpu

IN_DIM = 1024
OUT_DIM = 1024
N_PROTO = 64
RANK = 16
TOK_BLK = 512


def _body(x_ref, pt_ref, bt_ref, af_ref, bias_ref, temp_ref, o_ref):
    xb = x_ref[...]
    pt = pt_ref[...]
    logits = jnp.dot(xb.astype(jnp.bfloat16), pt.astype(jnp.bfloat16),
                     preferred_element_type=jnp.float32)
    x2 = jnp.sum(xb * xb, axis=1, keepdims=True)
    p2 = jnp.sum(pt * pt, axis=0, keepdims=True)
    d2 = jnp.maximum(x2 + p2 - 2.0 * logits, 0.0)
    d = jnp.sqrt(d2)
    t = jnp.maximum(jnp.abs(temp_ref[0, 0]), 0.1)
    s = -d / t

    iota = jax.lax.broadcasted_iota(jnp.int32, s.shape, 1)
    m1 = jnp.max(s, axis=1, keepdims=True)
    i1 = jnp.min(jnp.where(s == m1, iota, N_PROTO), axis=1, keepdims=True)
    s_excl = jnp.where(iota == i1, -jnp.inf, s)
    m2 = jnp.max(s_excl, axis=1, keepdims=True)
    i2 = jnp.min(jnp.where(s_excl == m2, iota, N_PROTO), axis=1, keepdims=True)
    e = jnp.exp(m2 - m1)
    w1 = 1.0 / (1.0 + e)
    w2 = 1.0 - w1

    M = jnp.where(iota == i1, w1, 0.0) + jnp.where(iota == i2, w2, 0.0)

    H = jnp.dot(xb, bt_ref[...], preferred_element_type=jnp.float32)
    lane_e = jax.lax.broadcasted_iota(jnp.int32, H.shape, 1) // RANK
    G = H * (jnp.where(lane_e == i1, w1, 0.0) + jnp.where(lane_e == i2, w2, 0.0))

    out = jnp.dot(G, af_ref[...], preferred_element_type=jnp.float32)
    out = out + jnp.dot(M, bias_ref[...], preferred_element_type=jnp.float32)
    o_ref[...] = out


def kernel(x, prototypes, B, A, bias, temp):
    lead_shape = x.shape[:-1]
    xf = x.reshape(-1, x.shape[-1])
    n_tok = xf.shape[0]

    pt = prototypes.T
    bt = B.reshape(N_PROTO * RANK, IN_DIM).T
    af = A.transpose(0, 2, 1).reshape(N_PROTO * RANK, OUT_DIM)
    temp_arr = jnp.asarray(temp, jnp.float32).reshape(1, 1)

    grid = (n_tok // TOK_BLK,)
    out = pl.pallas_call(
        _body,
        grid=grid,
        in_specs=[
            pl.BlockSpec((TOK_BLK, IN_DIM), lambda i: (i, 0)),
            pl.BlockSpec((IN_DIM, N_PROTO), lambda i: (0, 0)),
            pl.BlockSpec((IN_DIM, N_PROTO * RANK), lambda i: (0, 0)),
            pl.BlockSpec((N_PROTO * RANK, OUT_DIM), lambda i: (0, 0)),
            pl.BlockSpec((N_PROTO, OUT_DIM), lambda i: (0, 0)),
            pl.BlockSpec((1, 1), lambda i: (0, 0)),
        ],
        out_specs=pl.BlockSpec((TOK_BLK, OUT_DIM), lambda i: (i, 0)),
        out_shape=jax.ShapeDtypeStruct((n_tok, OUT_DIM), jnp.float32),
    )(xf, pt, bt, af, bias, temp_arr)
    return out.reshape(*lead_shape, OUT_DIM)


# speedup vs baseline: 20.4085x; 1.3625x over previous
import jax
import jax.numpy as jnp
from jax.experimental import pallas as pl
from jax.experimental.pallas import tpu as pltpu

IN_DIM = 1024
OUT_DIM = 1024
N_PROTO = 64
RANK = 16
TOK_BLK = 512


def _body(x_ref, pt_ref, bt_ref, af_ref, bias_ref, temp_ref, o_ref):
    xb = x_ref[...]
    xb16 = xb.astype(jnp.bfloat16)
    pt = pt_ref[...]
    logits = jnp.dot(xb16, pt.astype(jnp.bfloat16),
                     preferred_element_type=jnp.float32)
    x2 = jnp.sum(xb * xb, axis=1, keepdims=True)
    p2 = jnp.sum(pt * pt, axis=0, keepdims=True)
    d2 = jnp.maximum(x2 + p2 - 2.0 * logits, 0.0)
    d = jnp.sqrt(d2)
    t = jnp.maximum(jnp.abs(temp_ref[0, 0]), 0.1)
    s = -d / t

    iota = jax.lax.broadcasted_iota(jnp.int32, s.shape, 1)
    m1 = jnp.max(s, axis=1, keepdims=True)
    i1 = jnp.min(jnp.where(s == m1, iota, N_PROTO), axis=1, keepdims=True)
    s_excl = jnp.where(iota == i1, -jnp.inf, s)
    m2 = jnp.max(s_excl, axis=1, keepdims=True)
    i2 = jnp.min(jnp.where(s_excl == m2, iota, N_PROTO), axis=1, keepdims=True)
    e = jnp.exp(m2 - m1)
    w1 = 1.0 / (1.0 + e)
    w2 = 1.0 - w1

    M = jnp.where(iota == i1, w1, 0.0) + jnp.where(iota == i2, w2, 0.0)

    H = jnp.dot(xb16, bt_ref[...], preferred_element_type=jnp.float32)
    lane_e = jax.lax.broadcasted_iota(jnp.int32, H.shape, 1) // RANK
    G = H * (jnp.where(lane_e == i1, w1, 0.0) + jnp.where(lane_e == i2, w2, 0.0))

    out = jnp.dot(G.astype(jnp.bfloat16), af_ref[...],
                  preferred_element_type=jnp.float32)
    out = out + jnp.dot(M, bias_ref[...], preferred_element_type=jnp.float32)
    o_ref[...] = out


def kernel(x, prototypes, B, A, bias, temp):
    lead_shape = x.shape[:-1]
    xf = x.reshape(-1, x.shape[-1])
    n_tok = xf.shape[0]

    pt = prototypes.T
    bt = B.reshape(N_PROTO * RANK, IN_DIM).T.astype(jnp.bfloat16)
    af = A.transpose(0, 2, 1).reshape(N_PROTO * RANK, OUT_DIM).astype(jnp.bfloat16)
    temp_arr = jnp.asarray(temp, jnp.float32).reshape(1, 1)

    grid = (n_tok // TOK_BLK,)
    out = pl.pallas_call(
        _body,
        grid=grid,
        in_specs=[
            pl.BlockSpec((TOK_BLK, IN_DIM), lambda i: (i, 0)),
            pl.BlockSpec((IN_DIM, N_PROTO), lambda i: (0, 0)),
            pl.BlockSpec((IN_DIM, N_PROTO * RANK), lambda i: (0, 0)),
            pl.BlockSpec((N_PROTO * RANK, OUT_DIM), lambda i: (0, 0)),
            pl.BlockSpec((N_PROTO, OUT_DIM), lambda i: (0, 0)),
            pl.BlockSpec((1, 1), lambda i: (0, 0)),
        ],
        out_specs=pl.BlockSpec((TOK_BLK, OUT_DIM), lambda i: (i, 0)),
        out_shape=jax.ShapeDtypeStruct((n_tok, OUT_DIM), jnp.float32),
    )(xf, pt, bt, af, bias, temp_arr)
    return out.reshape(*lead_shape, OUT_DIM)
